# baseline (speedup 1.0000x reference)
import jax
import jax.numpy as jnp
from jax import lax
from jax.experimental import pallas as pl
from jax.experimental.pallas import tpu as pltpu
from jax.experimental.pallas import tpu_sc as plsc

DIM = 64
BATCH = 4096
HIST = 50
PRETRAINED = 100000
NC, NS = 2, 16
NW = NC * NS
BPT = BATCH // NW
CHUNK = BPT
NBUF = 5
AHEAD = NBUF - 1
NOUTER = HIST // NBUF
NG = CHUNK // 16


def _gather_body(idx_hbm, krows_hbm, table_hbm, out_hbm,
                 idx_v, sidx_v, kv, bufs, tbuf, minv, gsem):
    wid = lax.axis_index("s") * NC + lax.axis_index("c")
    b0 = wid * BPT
    pltpu.sync_copy(idx_hbm.at[:, pl.ds(b0, BPT)], idx_v)
    pltpu.sync_copy(krows_hbm, kv)

    @plsc.parallel_loop(0, HIST, unroll=2)
    def _prep(h):
        acc = None
        for g in range(NG):
            sl = pl.ds(g * 16, 16)
            v = idx_v[h, sl]
            acc = v if acc is None else jnp.minimum(acc, v)
            sidx_v[h, sl] = jnp.maximum(v - 2, 0)
        minv[pl.ds(h * 16, 16)] = acc

    def fire(h, b):
        pltpu.async_copy(table_hbm.at[sidx_v.at[h]], bufs.at[b], gsem.at[b])

    for hp in range(AHEAD):
        fire(hp, hp)

    iota = lax.iota(jnp.int32, 16)
    jhi = [(iota + jg * 16) // 8 for jg in range(DIM // 16)]
    jlo = [(iota + jg * 16) % 8 for jg in range(DIM // 16)]

    def body(h0, carry):
        for b in range(NBUF):
            h = h0 * NBUF + b
            pltpu.make_async_copy(
                table_hbm.at[sidx_v.at[h]], bufs.at[b], gsem.at[b],
            ).wait()
            br = bufs.at[b]

            mv = minv[pl.ds(h * 16, 16)]
            mmin = mv[0]
            for l in range(1, 16):
                mmin = jnp.minimum(mmin, mv[l])

            @pl.when(mmin < 2)
            def _(h=h, br=br):
                def g_body(g, carry2):
                    v = idx_v[h, pl.ds(g * 16, 16)]
                    for l in range(16):
                        s = v[l]

                        @pl.when(s < 2)
                        def __(s=s, l=l, g=g):
                            for c in range(DIM // 16):
                                sl = pl.ds(c * 16, 16)
                                br[g * 16 + l, sl] = kv[s, sl]
                    return carry2

                lax.fori_loop(0, NG, g_body, 0, unroll=False)

            @plsc.parallel_loop(0, CHUNK, unroll=16)
            def _tr(bcol, br=br):
                col = jnp.zeros((16,), jnp.int32) + bcol
                for jg in range(DIM // 16):
                    vals = br[bcol, pl.ds(jg * 16, 16)]
                    plsc.store_scatter(tbuf, [jhi[jg], jlo[jg], col], vals)

            pltpu.sync_copy(tbuf.at[:, :, pl.ds(0, CHUNK)],
                            out_hbm.at[h, :, wid, :, :])
            k = h + AHEAD
            bk = (b + AHEAD) % NBUF

            @pl.when(k < HIST)
            def _(k=k, bk=bk):
                fire(k, bk)
        return carry

    lax.fori_loop(0, NOUTER, body, 0, unroll=False)


def kernel(inputs, kernel, pretrained):
    idx = inputs.T.astype(jnp.int32)
    mesh = plsc.VectorSubcoreMesh(core_axis_name="c", subcore_axis_name="s")
    out = pl.kernel(
        _gather_body,
        mesh=mesh,
        compiler_params=pltpu.CompilerParams(
            use_tc_tiling_on_sc=False, needs_layout_passes=False,
            disable_bounds_checks=True, skip_device_barrier=True),
        out_type=jax.ShapeDtypeStruct((HIST, DIM // 8, NW, 8, BPT), jnp.float32),
        scratch_types=[
            pltpu.VMEM((HIST, BPT), jnp.int32),
            pltpu.VMEM((HIST, BPT), jnp.int32),
            pltpu.VMEM((2, DIM), jnp.float32),
            pltpu.VMEM((NBUF, CHUNK, DIM), jnp.float32),
            pltpu.VMEM((DIM // 8, 8, CHUNK + 1), jnp.float32),
            pltpu.VMEM((HIST * 16,), jnp.int32),
            pltpu.SemaphoreType.DMA((NBUF,)),
        ],
    )(idx, kernel, pretrained)
    return out.transpose(2, 4, 0, 1, 3).reshape(BATCH, HIST, DIM)


# speedup vs baseline: 1.0865x; 1.0865x over previous
import jax
import jax.numpy as jnp
from jax import lax
from jax.experimental import pallas as pl
from jax.experimental.pallas import tpu as pltpu
from jax.experimental.pallas import tpu_sc as plsc

DIM = 64
BATCH = 4096
HIST = 50
PRETRAINED = 100000
NC, NS = 2, 16
NW = NC * NS
BPT = BATCH // NW
CHUNK = BPT
NBUF = 5
AHEAD = NBUF - 1
NOUTER = HIST // NBUF
NG = CHUNK // 16


def _gather_body(idx_hbm, krows_hbm, table_hbm, out_hbm,
                 idx_v, sidx_v, kv, bufs, tbuf, minv, gsem):
    wid = lax.axis_index("s") * NC + lax.axis_index("c")
    b0 = wid * BPT
    pltpu.sync_copy(idx_hbm.at[:, pl.ds(b0, BPT)], idx_v)
    pltpu.sync_copy(krows_hbm, kv)

    @plsc.parallel_loop(0, HIST, unroll=2)
    def _prep(h):
        acc = None
        for g in range(NG):
            sl = pl.ds(g * 16, 16)
            v = idx_v[h, sl]
            acc = v if acc is None else jnp.minimum(acc, v)
            sidx_v[h, sl] = jnp.maximum(v - 2, 0)
        minv[pl.ds(h * 16, 16)] = acc

    def fire(h, b):
        pltpu.async_copy(table_hbm.at[sidx_v.at[h]], bufs.at[b], gsem.at[b])

    for hp in range(AHEAD):
        fire(hp, hp)

    iota = lax.iota(jnp.int32, 16)
    jhi = [(iota + jg * 16) // 8 for jg in range(DIM // 16)]
    jlo = [(iota + jg * 16) % 8 for jg in range(DIM // 16)]

    def body(h0, carry):
        for b in range(NBUF):
            h = h0 * NBUF + b
            pltpu.make_async_copy(
                table_hbm.at[sidx_v.at[h]], bufs.at[b], gsem.at[b],
            ).wait()
            br = bufs.at[b]

            mv = minv[pl.ds(h * 16, 16)]
            mmin = mv[0]
            for l in range(1, 16):
                mmin = jnp.minimum(mmin, mv[l])

            @pl.when(mmin < 2)
            def _(h=h, br=br):
                def g_body(g, carry2):
                    v = idx_v[h, pl.ds(g * 16, 16)]
                    for l in range(16):
                        s = v[l]

                        @pl.when(s < 2)
                        def __(s=s, l=l, g=g):
                            for c in range(DIM // 16):
                                sl = pl.ds(c * 16, 16)
                                br[g * 16 + l, sl] = kv[s, sl]
                    return carry2

                lax.fori_loop(0, NG, g_body, 0, unroll=False)

            @plsc.parallel_loop(0, CHUNK, unroll=8)
            def _tr(bcol, br=br):
                col = jnp.zeros((16,), jnp.int32) + bcol
                for jg in range(DIM // 16):
                    vals = br[bcol, pl.ds(jg * 16, 16)]
                    plsc.store_scatter(tbuf, [jhi[jg], jlo[jg], col], vals)

            pltpu.sync_copy(tbuf.at[:, :, pl.ds(0, CHUNK)],
                            out_hbm.at[h, :, wid, :, :])
            k = h + AHEAD
            bk = (b + AHEAD) % NBUF

            @pl.when(k < HIST)
            def _(k=k, bk=bk):
                fire(k, bk)
        return carry

    lax.fori_loop(0, NOUTER, body, 0, unroll=False)


def kernel(inputs, kernel, pretrained):
    idx = inputs.T.astype(jnp.int32)
    mesh = plsc.VectorSubcoreMesh(core_axis_name="c", subcore_axis_name="s")
    out = pl.kernel(
        _gather_body,
        mesh=mesh,
        compiler_params=pltpu.CompilerParams(
            use_tc_tiling_on_sc=False, needs_layout_passes=False,
            disable_bounds_checks=True, skip_device_barrier=True),
        out_type=jax.ShapeDtypeStruct((HIST, DIM // 8, NW, 8, BPT), jnp.float32),
        scratch_types=[
            pltpu.VMEM((HIST, BPT), jnp.int32),
            pltpu.VMEM((HIST, BPT), jnp.int32),
            pltpu.VMEM((2, DIM), jnp.float32),
            pltpu.VMEM((NBUF, CHUNK, DIM), jnp.float32),
            pltpu.VMEM((DIM // 8, 8, CHUNK + 1), jnp.float32),
            pltpu.VMEM((HIST * 16,), jnp.int32),
            pltpu.SemaphoreType.DMA((NBUF,)),
        ],
    )(idx, kernel, pretrained)
    return out.transpose(2, 4, 0, 1, 3).reshape(BATCH, HIST, DIM)


# speedup vs baseline: 1.1419x; 1.0510x over previous
import jax
import jax.numpy as jnp
from jax import lax
from jax.experimental import pallas as pl
from jax.experimental.pallas import tpu as pltpu
from jax.experimental.pallas import tpu_sc as plsc

DIM = 64
BATCH = 4096
HIST = 50
PRETRAINED = 100000
NC, NS = 2, 16
NW = NC * NS
BPT = BATCH // NW
CHUNK = BPT
NBUF = 5
AHEAD = NBUF - 1
NOUTER = HIST // NBUF
NG = CHUNK // 16


def _gather_body(idx_hbm, krows_hbm, table_hbm, out_hbm,
                 idx_v, sidx_v, kv, bufs, tbufs, minv, gsem, osem):
    wid = lax.axis_index("s") * NC + lax.axis_index("c")
    b0 = wid * BPT
    pltpu.sync_copy(idx_hbm.at[:, pl.ds(b0, BPT)], idx_v)
    pltpu.sync_copy(krows_hbm, kv)

    @plsc.parallel_loop(0, HIST, unroll=2)
    def _prep(h):
        acc = None
        for g in range(NG):
            sl = pl.ds(g * 16, 16)
            v = idx_v[h, sl]
            acc = v if acc is None else jnp.minimum(acc, v)
            sidx_v[h, sl] = jnp.maximum(v - 2, 0)
        minv[pl.ds(h * 16, 16)] = acc

    def fire(h, b):
        pltpu.async_copy(table_hbm.at[sidx_v.at[h]], bufs.at[b], gsem.at[b])

    for hp in range(AHEAD):
        fire(hp, hp)

    iota = lax.iota(jnp.int32, 16)
    jhi = [(iota + jg * 16) // 8 for jg in range(DIM // 16)]
    jlo = [(iota + jg * 16) % 8 for jg in range(DIM // 16)]

    def body(h0, carry):
        for b in range(NBUF):
            h = h0 * NBUF + b
            pltpu.make_async_copy(
                table_hbm.at[sidx_v.at[h]], bufs.at[b], gsem.at[b],
            ).wait()
            br = bufs.at[b]

            mv = minv[pl.ds(h * 16, 16)]
            mmin = mv[0]
            for l in range(1, 16):
                mmin = jnp.minimum(mmin, mv[l])

            @pl.when(mmin < 2)
            def _(h=h, br=br):
                def g_body(g, carry2):
                    v = idx_v[h, pl.ds(g * 16, 16)]
                    for l in range(16):
                        s = v[l]

                        @pl.when(s < 2)
                        def __(s=s, l=l, g=g):
                            for c in range(DIM // 16):
                                sl = pl.ds(c * 16, 16)
                                br[g * 16 + l, sl] = kv[s, sl]
                    return carry2

                lax.fori_loop(0, NG, g_body, 0, unroll=False)

            tb = b % 2
            tbuf = tbufs.at[tb]

            @pl.when(h >= 2)
            def _(h=h, tb=tb, tbuf=tbuf):
                pltpu.make_async_copy(
                    tbuf.at[:, :, pl.ds(0, CHUNK)],
                    out_hbm.at[h - 2, :, wid, :, :], osem.at[tb],
                ).wait()

            @plsc.parallel_loop(0, CHUNK, unroll=8)
            def _tr(bcol, br=br, tbuf=tbuf):
                col = jnp.zeros((16,), jnp.int32) + bcol
                for jg in range(DIM // 16):
                    vals = br[bcol, pl.ds(jg * 16, 16)]
                    plsc.store_scatter(tbuf, [jhi[jg], jlo[jg], col], vals)

            pltpu.async_copy(tbuf.at[:, :, pl.ds(0, CHUNK)],
                             out_hbm.at[h, :, wid, :, :], osem.at[tb])
            k = h + AHEAD
            bk = (b + AHEAD) % NBUF

            @pl.when(k < HIST)
            def _(k=k, bk=bk):
                fire(k, bk)
        return carry

    lax.fori_loop(0, NOUTER, body, 0, unroll=False)

    for hh, tb in ((HIST - 2, (NBUF - 2) % 2), (HIST - 1, (NBUF - 1) % 2)):
        pltpu.make_async_copy(
            tbufs.at[tb].at[:, :, pl.ds(0, CHUNK)],
            out_hbm.at[hh, :, wid, :, :], osem.at[tb],
        ).wait()


def kernel(inputs, kernel, pretrained):
    idx = inputs.T.astype(jnp.int32)
    mesh = plsc.VectorSubcoreMesh(core_axis_name="c", subcore_axis_name="s")
    out = pl.kernel(
        _gather_body,
        mesh=mesh,
        compiler_params=pltpu.CompilerParams(
            use_tc_tiling_on_sc=False, needs_layout_passes=False,
            disable_bounds_checks=True, skip_device_barrier=True),
        out_type=jax.ShapeDtypeStruct((HIST, DIM // 8, NW, 8, BPT), jnp.float32),
        scratch_types=[
            pltpu.VMEM((HIST, BPT), jnp.int32),
            pltpu.VMEM((HIST, BPT), jnp.int32),
            pltpu.VMEM((2, DIM), jnp.float32),
            pltpu.VMEM((NBUF, CHUNK, DIM), jnp.float32),
            pltpu.VMEM((2, DIM // 8, 8, CHUNK + 1), jnp.float32),
            pltpu.VMEM((HIST * 16,), jnp.int32),
            pltpu.SemaphoreType.DMA((NBUF,)),
            pltpu.SemaphoreType.DMA((2,)),
        ],
    )(idx, kernel, pretrained)
    return out.transpose(2, 4, 0, 1, 3).reshape(BATCH, HIST, DIM)
